# baseline (speedup 1.0000x reference)
import functools

import jax
import jax.numpy as jnp
from jax import lax
from jax.experimental import pallas as pl
from jax.experimental.pallas import tpu as pltpu
from jax.experimental.pallas import tpu_sc as plsc

TOTAL_TOKENS = 32768
RETAIN = 8192
DROP = TOTAL_TOKENS - RETAIN
BATCH = 128

_NC = 2
_NS = 16
_NW = _NC * _NS
_ROWS_PER_W = BATCH // _NW
_L = 16
_CHUNK = 8192
_NCHUNK = TOTAL_TOKENS // _CHUNK
_NOUT = 3
_PSLICE = TOTAL_TOKENS // _NS


@functools.partial(
    pl.kernel,
    mesh=plsc.VectorSubcoreMesh(core_axis_name="c", subcore_axis_name="s"),
    compiler_params=pltpu.CompilerParams(needs_layout_passes=False),
    out_type=(
        jax.ShapeDtypeStruct((BATCH, RETAIN), jnp.float32),
        jax.ShapeDtypeStruct((BATCH, DROP), jnp.float32),
    ),
    scratch_types=[
        pltpu.VMEM((TOTAL_TOKENS,), jnp.int32),
        pltpu.VMEM((TOTAL_TOKENS,), jnp.float32),
        pltpu.VMEM((TOTAL_TOKENS,), jnp.float32),
        pltpu.VMEM((_CHUNK,), jnp.float32),
        pltpu.VMEM((_CHUNK,), jnp.float32),
        pltpu.VMEM((_CHUNK,), jnp.float32),
        pltpu.VMEM_SHARED((TOTAL_TOKENS,), jnp.int32),
        pltpu.SemaphoreType.DMA,
        pltpu.SemaphoreType.DMA,
        pltpu.SemaphoreType.DMA,
        pltpu.SemaphoreType.DMA,
        pltpu.SemaphoreType.DMA,
        pltpu.SemaphoreType.DMA,
    ],
)
def _sampler(x_hbm, perm_hbm, y_hbm, z_hbm, perm_v, row0_v, row1_v,
             o0_v, o1_v, o2_v, perm_s, sem_perm, sem_r0, sem_r1,
             so0, so1, so2):
    sid = lax.axis_index("s")
    wid = sid * _NC + lax.axis_index("c")
    base = wid * _ROWS_PER_W
    rows = (row0_v, row1_v)
    row_sems = (sem_r0, sem_r1)
    outs = (o0_v, o1_v, o2_v)
    out_sems = (so0, so1, so2)

    psl = pl.ds(sid * _PSLICE, _PSLICE)
    pltpu.sync_copy(perm_hbm.at[psl], perm_s.at[psl])
    row_cp = [None, None]
    row_cp[0] = pltpu.async_copy(x_hbm.at[base], row0_v, sem_r0)
    plsc.subcore_barrier()
    pltpu.sync_copy(perm_s, perm_v)

    out_cp = [None] * _NOUT
    for r in range(_ROWS_PER_W):
        rb = r % 2
        row_cp[rb].wait()
        if r + 1 < _ROWS_PER_W:
            nb = (r + 1) % 2
            row_cp[nb] = pltpu.async_copy(
                x_hbm.at[base + r + 1], rows[nb], row_sems[nb])
        row_v = rows[rb]
        for c in range(_NCHUNK):
            g = r * _NCHUNK + c
            ob = g % _NOUT
            if out_cp[ob] is not None:
                out_cp[ob].wait()
            out_v = outs[ob]

            @plsc.parallel_loop(0, _CHUNK, step=_L, unroll=16)
            def _gather(j):
                idx = perm_v[pl.ds(c * _CHUNK + j, _L)]
                out_v[pl.ds(j, _L)] = plsc.load_gather(row_v, [idx])

            if c == 0:
                dst = y_hbm.at[base + r]
            else:
                dst = z_hbm.at[base + r, pl.ds((c - 1) * _CHUNK, _CHUNK)]
            out_cp[ob] = pltpu.async_copy(out_v, dst, out_sems[ob])
    for cp in out_cp:
        cp.wait()


def kernel(x, perm):
    return _sampler(x, perm.astype(jnp.int32))


# speedup vs baseline: 1.0299x; 1.0299x over previous
import functools

import jax
import jax.numpy as jnp
from jax import lax
from jax.experimental import pallas as pl
from jax.experimental.pallas import tpu as pltpu
from jax.experimental.pallas import tpu_sc as plsc

TOTAL_TOKENS = 32768
RETAIN = 8192
DROP = TOTAL_TOKENS - RETAIN
BATCH = 128

_NC = 2
_NS = 16
_NW = _NC * _NS
_ROWS_PER_W = BATCH // _NW
_L = 16
_CHUNK = 8192
_NCHUNK = TOTAL_TOKENS // _CHUNK
_NOUT = 3
_PSLICE = TOTAL_TOKENS // _NS


@functools.partial(
    pl.kernel,
    mesh=plsc.VectorSubcoreMesh(core_axis_name="c", subcore_axis_name="s"),
    compiler_params=pltpu.CompilerParams(needs_layout_passes=False),
    out_type=(
        jax.ShapeDtypeStruct((BATCH, RETAIN), jnp.float32),
        jax.ShapeDtypeStruct((BATCH, DROP), jnp.float32),
    ),
    scratch_types=[
        pltpu.VMEM((TOTAL_TOKENS,), jnp.int32),
        pltpu.VMEM((TOTAL_TOKENS,), jnp.float32),
        pltpu.VMEM((TOTAL_TOKENS,), jnp.float32),
        pltpu.VMEM((_CHUNK,), jnp.float32),
        pltpu.VMEM((_CHUNK,), jnp.float32),
        pltpu.VMEM((_CHUNK,), jnp.float32),
        pltpu.VMEM_SHARED((TOTAL_TOKENS,), jnp.int32),
        pltpu.SemaphoreType.DMA,
        pltpu.SemaphoreType.DMA,
        pltpu.SemaphoreType.DMA,
        pltpu.SemaphoreType.DMA,
        pltpu.SemaphoreType.DMA,
        pltpu.SemaphoreType.DMA,
    ],
)
def _sampler(x_hbm, perm_hbm, y_hbm, z_hbm, perm_v, row0_v, row1_v,
             o0_v, o1_v, o2_v, perm_s, sem_perm, sem_r0, sem_r1,
             so0, so1, so2):
    sid = lax.axis_index("s")
    wid = sid * _NC + lax.axis_index("c")
    base = wid * _ROWS_PER_W
    rows = (row0_v, row1_v)
    row_sems = (sem_r0, sem_r1)
    outs = (o0_v, o1_v, o2_v)
    out_sems = (so0, so1, so2)

    psl = pl.ds(sid * _PSLICE, _PSLICE)
    pltpu.sync_copy(perm_hbm.at[psl], perm_s.at[psl])
    row_cp = [None, None]
    row_cp[0] = pltpu.async_copy(x_hbm.at[base], row0_v, sem_r0)
    plsc.subcore_barrier()
    pltpu.sync_copy(perm_s, perm_v)

    out_cp = [None] * _NOUT
    for r in range(_ROWS_PER_W):
        rb = r % 2
        row_cp[rb].wait()
        if r + 1 < _ROWS_PER_W:
            nb = (r + 1) % 2
            row_cp[nb] = pltpu.async_copy(
                x_hbm.at[base + r + 1], rows[nb], row_sems[nb])
        row_v = rows[rb]
        for c in range(_NCHUNK):
            g = r * _NCHUNK + c
            ob = g % _NOUT
            if out_cp[ob] is not None:
                out_cp[ob].wait()
            out_v = outs[ob]

            @plsc.parallel_loop(0, _CHUNK, step=_L, unroll=8)
            def _gather(j):
                idx = perm_v[pl.ds(c * _CHUNK + j, _L)]
                out_v[pl.ds(j, _L)] = plsc.load_gather(row_v, [idx])

            if c == 0:
                dst = y_hbm.at[base + r]
            else:
                dst = z_hbm.at[base + r, pl.ds((c - 1) * _CHUNK, _CHUNK)]
            out_cp[ob] = pltpu.async_copy(out_v, dst, out_sems[ob])
    for cp in out_cp:
        cp.wait()


def kernel(x, perm):
    return _sampler(x, perm.astype(jnp.int32))
